# baseline (speedup 1.0000x reference)
import functools

import jax
import jax.numpy as jnp
from jax import lax
from jax.experimental import pallas as pl
from jax.experimental.pallas import tpu as pltpu
from jax.experimental.pallas import tpu_sc as plsc

N = 10000
E = 320000
D_IN = 128
D_H = 128
D_OUT = 64

NC = 2
NS = 16
NW = NC * NS

SUB = 128
KROWS = 8
ROWS_PER_TILE = 80
OUTER = ROWS_PER_TILE // KROWS
EPAD = NW * ROWS_PER_TILE * SUB
EROWS = EPAD // SUB
NPAD = EPAD // NW
L = 16

@functools.cache
def _get_deg_pass():
    mesh = plsc.VectorSubcoreMesh(core_axis_name="c", subcore_axis_name="s")
    rz = NPAD // NS

    @functools.partial(
        pl.kernel,
        out_type=jax.ShapeDtypeStruct((NC, NPAD), jnp.float32),
        mesh=mesh,
        scratch_types=[
            pltpu.VMEM((ROWS_PER_TILE, SUB), jnp.int32),
            pltpu.VMEM((NPAD,), jnp.float32),
            pltpu.VMEM((NS, rz), jnp.float32),
            pltpu.VMEM((rz,), jnp.float32),
            pltpu.VMEM_SHARED((NS, NPAD), jnp.float32),
        ],
        compiler_params=pltpu.CompilerParams(needs_layout_passes=False),
    )
    def _deg_pass(dst_hbm, out_hbm, dst_v, acc_v, red_v, out_v, sh):
        c = lax.axis_index("c")
        s = lax.axis_index("s")
        wid = c * NS + s
        row0 = wid * ROWS_PER_TILE
        pltpu.sync_copy(dst_hbm.at[pl.ds(row0, ROWS_PER_TILE)], dst_v)

        zeros = jnp.zeros((L,), jnp.float32)

        @pl.loop(0, NPAD, step=L)
        def _(j):
            acc_v[pl.ds(j, L)] = zeros

        ones = jnp.ones((L,), jnp.float32)

        @pl.loop(0, ROWS_PER_TILE)
        def _(r):
            for k in range(SUB // L):
                idx = dst_v[r, pl.ds(k * L, L)]
                plsc.addupdate_scatter(acc_v, [idx], ones)

        pltpu.sync_copy(acc_v, sh.at[s])
        plsc.subcore_barrier()
        for r in range(NS):
            pltpu.sync_copy(sh.at[r, pl.ds(s * rz, rz)], red_v.at[r])

        @pl.loop(0, rz, step=L)
        def _(j):
            v = red_v[0, pl.ds(j, L)]
            for r in range(1, NS):
                v = v + red_v[r, pl.ds(j, L)]
            out_v[pl.ds(j, L)] = v

        pltpu.sync_copy(out_v, out_hbm.at[c, pl.ds(s * rz, rz)])

    return _deg_pass


NBUF = 2
STAGE = 40
EDGE_CORES = 2
EDGE_RPT = EROWS // (EDGE_CORES * NS)


@functools.cache
def _make_edge_pass(D, ncores=NC, rows_per_tile=ROWS_PER_TILE):
    mesh = plsc.VectorSubcoreMesh(
        core_axis_name="c", subcore_axis_name="s", num_cores=ncores)

    @functools.partial(
        pl.kernel,
        out_type=jax.ShapeDtypeStruct((ncores, NPAD, D), jnp.float32),
        mesh=mesh,
        scratch_types=[
            pltpu.VMEM((STAGE, SUB), jnp.int32),
            pltpu.VMEM((STAGE, SUB), jnp.int32),
        ]
        + [pltpu.VMEM((SUB, D), jnp.float32) for _ in range(NBUF)]
        + [pltpu.SemaphoreType.DMA for _ in range(2 * NBUF)]
        + [pltpu.VMEM_SHARED((NPAD, D), jnp.float32)],
        compiler_params=pltpu.CompilerParams(use_tc_tiling_on_sc=False),
    )
    def edge_pass(hs_hbm, src_hbm, dst_hbm, zeros_hbm, out_hbm,
                  src_v, dst_v, *rest):
        bufs = rest[:NBUF]
        sg = rest[NBUF:2 * NBUF]
        ss = rest[2 * NBUF:3 * NBUF]
        acc = rest[3 * NBUF]
        c = lax.axis_index("c")
        s = lax.axis_index("s")
        wid = c * NS + s
        rz = NPAD // NS
        pltpu.sync_copy(zeros_hbm.at[pl.ds(s * rz, rz)],
                        acc.at[pl.ds(s * rz, rz)])
        plsc.subcore_barrier()
        row0 = wid * rows_per_tile

        @pl.loop(0, rows_per_tile // STAGE)
        def _(h):
            r0 = row0 + h * STAGE
            pltpu.sync_copy(src_hbm.at[pl.ds(r0, STAGE)], src_v)
            pltpu.sync_copy(dst_hbm.at[pl.ds(r0, STAGE)], dst_v)
            for b in range(NBUF):
                pltpu.async_copy(hs_hbm.at[src_v.at[b]], bufs[b], sg[b])

            @pl.loop(0, STAGE - NBUF, step=NBUF)
            def _(j):
                for b in range(NBUF):
                    pltpu.make_async_copy(
                        hs_hbm.at[src_v.at[j + b]], bufs[b], sg[b]).wait()
                    pltpu.async_copy(
                        bufs[b], acc.at[dst_v.at[j + b]], ss[b], add=True)
                for b in range(NBUF):
                    pltpu.make_async_copy(
                        bufs[b], acc.at[dst_v.at[j + b]], ss[b]).wait()
                    pltpu.async_copy(
                        hs_hbm.at[src_v.at[j + NBUF + b]], bufs[b], sg[b])

            j0 = STAGE - NBUF
            for b in range(NBUF):
                pltpu.make_async_copy(
                    hs_hbm.at[src_v.at[j0 + b]], bufs[b], sg[b]).wait()
                pltpu.async_copy(
                    bufs[b], acc.at[dst_v.at[j0 + b]], ss[b], add=True)
            for b in range(NBUF):
                pltpu.make_async_copy(
                    bufs[b], acc.at[dst_v.at[j0 + b]], ss[b]).wait()

        plsc.subcore_barrier()
        pltpu.sync_copy(acc.at[pl.ds(s * rz, rz)],
                        out_hbm.at[c, pl.ds(s * rz, rz)])

    return edge_pass


_R = 1000


def _dinv_col(degp_ref):
    p = degp_ref[0] + degp_ref[1]
    return lax.rsqrt(1.0 + p)


def _tc_hs1_body(x_ref, w_ref, degp_ref, o_ref):
    dinv = _dinv_col(degp_ref)
    o_ref[...] = dinv * jnp.dot(x_ref[...], w_ref[...],
                                preferred_element_type=jnp.float32)


def _tc_mid_body(aggp_ref, hs_ref, degp_ref, b_ref, w_ref, o_ref):
    dinv = _dinv_col(degp_ref)
    agg = jnp.sum(aggp_ref[...], axis=0)
    t = dinv * (agg + hs_ref[...]) + b_ref[...]
    out1 = jnp.maximum(t, 0.0)
    o_ref[...] = dinv * jnp.dot(out1, w_ref[...],
                                preferred_element_type=jnp.float32)


def _tc_fin_body(aggp_ref, hs_ref, degp_ref, b_ref, o_ref):
    dinv = _dinv_col(degp_ref)
    agg = jnp.sum(aggp_ref[...], axis=0)[:, :D_OUT]
    z = dinv * (agg + hs_ref[...]) + b_ref[...]
    m = jnp.max(z, axis=1, keepdims=True)
    e = jnp.exp(z - m)
    lse = jnp.log(jnp.sum(e, axis=1, keepdims=True)) + m
    o_ref[...] = z - lse


def _tc_hs1(x, W1, degp):
    return pl.pallas_call(
        _tc_hs1_body,
        grid=(N // _R,),
        in_specs=[
            pl.BlockSpec((_R, D_IN), lambda i: (i, 0)),
            pl.BlockSpec((D_IN, D_H), lambda i: (0, 0)),
            pl.BlockSpec((NC, _R, 1), lambda i: (0, i, 0)),
        ],
        out_specs=pl.BlockSpec((_R, D_H), lambda i: (i, 0)),
        out_shape=jax.ShapeDtypeStruct((N, D_H), jnp.float32),
    )(x, W1, degp)


def _tc_mid(agg1p, hs1, degp, b1, W2):
    return pl.pallas_call(
        _tc_mid_body,
        grid=(N // _R,),
        in_specs=[
            pl.BlockSpec((agg1p.shape[0], _R, D_H), lambda i: (0, i, 0)),
            pl.BlockSpec((_R, D_H), lambda i: (i, 0)),
            pl.BlockSpec((NC, _R, 1), lambda i: (0, i, 0)),
            pl.BlockSpec((1, D_H), lambda i: (0, 0)),
            pl.BlockSpec((D_H, D_OUT), lambda i: (0, 0)),
        ],
        out_specs=pl.BlockSpec((_R, D_OUT), lambda i: (i, 0)),
        out_shape=jax.ShapeDtypeStruct((N, D_OUT), jnp.float32),
    )(agg1p, hs1, degp, b1, W2)


def _tc_fin(agg2p, hs2, degp, b2):
    return pl.pallas_call(
        _tc_fin_body,
        grid=(N // _R,),
        in_specs=[
            pl.BlockSpec((agg2p.shape[0], _R, agg2p.shape[2]),
                         lambda i: (0, i, 0)),
            pl.BlockSpec((_R, D_OUT), lambda i: (i, 0)),
            pl.BlockSpec((NC, _R, 1), lambda i: (0, i, 0)),
            pl.BlockSpec((1, D_OUT), lambda i: (0, 0)),
        ],
        out_specs=pl.BlockSpec((_R, D_OUT), lambda i: (i, 0)),
        out_shape=jax.ShapeDtypeStruct((N, D_OUT), jnp.float32),
    )(agg2p, hs2, degp, b2)


def kernel(x, edge_index, W1, b1, W2, b2):
    src = edge_index[0].astype(jnp.int32)
    dst = edge_index[1].astype(jnp.int32)
    pad = jnp.full((EPAD - E,), N, jnp.int32)
    srcp = jnp.concatenate([src, pad]).reshape(EROWS, SUB)
    dstp = jnp.concatenate([dst, pad]).reshape(EROWS, SUB)

    zeros_h = jnp.zeros((NPAD, D_H), jnp.float32)

    degp = _get_deg_pass()(dstp)[:, :, None]

    hs1 = _tc_hs1(x, W1, degp)
    hs1p = jnp.concatenate(
        [hs1, jnp.zeros((NPAD - N, D_H), jnp.float32)], axis=0)
    agg1p = _make_edge_pass(D_H, EDGE_CORES, EDGE_RPT)(hs1p, srcp, dstp,
                                                       zeros_h)

    hs2 = _tc_mid(agg1p, hs1, degp, b1.reshape(1, D_H), W2)
    hs2p = jnp.concatenate(
        [hs2, jnp.zeros((NPAD - N, D_OUT), jnp.float32)], axis=0)
    agg2p = _make_edge_pass(D_OUT, EDGE_CORES, EDGE_RPT)(hs2p, srcp, dstp,
                                                         zeros_h[:, :D_OUT])

    return _tc_fin(agg2p, hs2, degp, b2.reshape(1, D_OUT))


# speedup vs baseline: 1.2326x; 1.2326x over previous
import functools

import jax
import jax.numpy as jnp
from jax import lax
from jax.experimental import pallas as pl
from jax.experimental.pallas import tpu as pltpu
from jax.experimental.pallas import tpu_sc as plsc

N = 10000
E = 320000
D_IN = 128
D_H = 128
D_OUT = 64

NC = 2
NS = 16
NW = NC * NS

SUB = 128
KROWS = 8
ROWS_PER_TILE = 80
OUTER = ROWS_PER_TILE // KROWS
EPAD = NW * ROWS_PER_TILE * SUB
EROWS = EPAD // SUB
NPAD = EPAD // NW
L = 16

@functools.cache
def _get_deg_pass():
    mesh = plsc.VectorSubcoreMesh(core_axis_name="c", subcore_axis_name="s")
    rz = NPAD // NS

    @functools.partial(
        pl.kernel,
        out_type=jax.ShapeDtypeStruct((NC, NPAD), jnp.float32),
        mesh=mesh,
        scratch_types=[
            pltpu.VMEM((ROWS_PER_TILE, SUB), jnp.int32),
            pltpu.VMEM((NPAD,), jnp.float32),
            pltpu.VMEM((NS, rz), jnp.float32),
            pltpu.VMEM((rz,), jnp.float32),
            pltpu.VMEM_SHARED((NS, NPAD), jnp.float32),
        ],
        compiler_params=pltpu.CompilerParams(needs_layout_passes=False),
    )
    def _deg_pass(dst_hbm, out_hbm, dst_v, acc_v, red_v, out_v, sh):
        c = lax.axis_index("c")
        s = lax.axis_index("s")
        wid = c * NS + s
        row0 = wid * ROWS_PER_TILE
        pltpu.sync_copy(dst_hbm.at[pl.ds(row0, ROWS_PER_TILE)], dst_v)

        zeros = jnp.zeros((L,), jnp.float32)

        @pl.loop(0, NPAD, step=L)
        def _(j):
            acc_v[pl.ds(j, L)] = zeros

        ones = jnp.ones((L,), jnp.float32)

        @pl.loop(0, ROWS_PER_TILE)
        def _(r):
            for k in range(SUB // L):
                idx = dst_v[r, pl.ds(k * L, L)]
                plsc.addupdate_scatter(acc_v, [idx], ones)

        pltpu.sync_copy(acc_v, sh.at[s])
        plsc.subcore_barrier()
        for r in range(NS):
            pltpu.sync_copy(sh.at[r, pl.ds(s * rz, rz)], red_v.at[r])

        @pl.loop(0, rz, step=L)
        def _(j):
            v = red_v[0, pl.ds(j, L)]
            for r in range(1, NS):
                v = v + red_v[r, pl.ds(j, L)]
            out_v[pl.ds(j, L)] = v

        pltpu.sync_copy(out_v, out_hbm.at[c, pl.ds(s * rz, rz)])

    return _deg_pass


NBUF = 2
STAGE = 40
EDGE_CORES = 2
EDGE_RPT = EROWS // (EDGE_CORES * NS)


@functools.cache
def _make_edge_pass(D, ncores=NC, rows_per_tile=ROWS_PER_TILE, nbuf=NBUF):
    mesh = plsc.VectorSubcoreMesh(
        core_axis_name="c", subcore_axis_name="s", num_cores=ncores)

    @functools.partial(
        pl.kernel,
        out_type=jax.ShapeDtypeStruct((ncores, NPAD, D), jnp.float32),
        mesh=mesh,
        scratch_types=[
            pltpu.VMEM((STAGE, SUB), jnp.int32),
            pltpu.VMEM((STAGE, SUB), jnp.int32),
        ]
        + [pltpu.VMEM((SUB, D), jnp.float32) for _ in range(nbuf)]
        + [pltpu.SemaphoreType.DMA for _ in range(2 * nbuf)]
        + [pltpu.VMEM_SHARED((NPAD, D), jnp.float32)],
        compiler_params=(
            pltpu.CompilerParams(use_tc_tiling_on_sc=False)
            if D % 128 != 0 else None),
    )
    def edge_pass(hs_hbm, src_hbm, dst_hbm, zeros_hbm, out_hbm,
                  src_v, dst_v, *rest):
        bufs = rest[:nbuf]
        sg = rest[nbuf:2 * nbuf]
        ss = rest[2 * nbuf:3 * nbuf]
        acc = rest[3 * nbuf]
        c = lax.axis_index("c")
        s = lax.axis_index("s")
        wid = c * NS + s
        rz = NPAD // NS
        pltpu.sync_copy(zeros_hbm.at[pl.ds(s * rz, rz)],
                        acc.at[pl.ds(s * rz, rz)])
        plsc.subcore_barrier()
        row0 = wid * rows_per_tile

        @pl.loop(0, rows_per_tile // STAGE)
        def _(h):
            r0 = row0 + h * STAGE
            pltpu.sync_copy(src_hbm.at[pl.ds(r0, STAGE)], src_v)
            pltpu.sync_copy(dst_hbm.at[pl.ds(r0, STAGE)], dst_v)
            for b in range(nbuf):
                pltpu.async_copy(hs_hbm.at[src_v.at[b]], bufs[b], sg[b])

            @pl.loop(0, STAGE - nbuf, step=nbuf)
            def _(j):
                for b in range(nbuf):
                    pltpu.make_async_copy(
                        hs_hbm.at[src_v.at[j + b]], bufs[b], sg[b]).wait()
                    pltpu.async_copy(
                        bufs[b], acc.at[dst_v.at[j + b]], ss[b], add=True)
                for b in range(nbuf):
                    pltpu.make_async_copy(
                        bufs[b], acc.at[dst_v.at[j + b]], ss[b]).wait()
                    pltpu.async_copy(
                        hs_hbm.at[src_v.at[j + nbuf + b]], bufs[b], sg[b])

            j0 = STAGE - nbuf
            for b in range(nbuf):
                pltpu.make_async_copy(
                    hs_hbm.at[src_v.at[j0 + b]], bufs[b], sg[b]).wait()
                pltpu.async_copy(
                    bufs[b], acc.at[dst_v.at[j0 + b]], ss[b], add=True)
            for b in range(nbuf):
                pltpu.make_async_copy(
                    bufs[b], acc.at[dst_v.at[j0 + b]], ss[b]).wait()

        plsc.subcore_barrier()
        pltpu.sync_copy(acc.at[pl.ds(s * rz, rz)],
                        out_hbm.at[c, pl.ds(s * rz, rz)])

    return edge_pass


NACC = 2


@functools.cache
def _make_edge_pass_bf16(D, nbuf=4):
    mesh = plsc.VectorSubcoreMesh(
        core_axis_name="c", subcore_axis_name="s", num_cores=NC)

    @functools.partial(
        pl.kernel,
        out_type=jax.ShapeDtypeStruct((NC, NACC, NPAD, D), jnp.bfloat16),
        mesh=mesh,
        scratch_types=[
            pltpu.VMEM((STAGE, SUB), jnp.int32),
            pltpu.VMEM((STAGE, SUB), jnp.int32),
        ]
        + [pltpu.VMEM((SUB, D), jnp.bfloat16) for _ in range(nbuf)]
        + [pltpu.SemaphoreType.DMA for _ in range(2 * nbuf)]
        + [pltpu.VMEM_SHARED((NPAD, D), jnp.bfloat16) for _ in range(NACC)],
        compiler_params=pltpu.CompilerParams(use_tc_tiling_on_sc=False),
    )
    def edge_pass(hs_hbm, src_hbm, dst_hbm, zeros_hbm, out_hbm,
                  src_v, dst_v, *rest):
        bufs = rest[:nbuf]
        sg = rest[nbuf:2 * nbuf]
        ss = rest[2 * nbuf:3 * nbuf]
        accs = rest[3 * nbuf:3 * nbuf + NACC]
        c = lax.axis_index("c")
        s = lax.axis_index("s")
        wid = c * NS + s
        rz = NPAD // NS
        for a in range(NACC):
            pltpu.sync_copy(zeros_hbm.at[pl.ds(s * rz, rz)],
                            accs[a].at[pl.ds(s * rz, rz)])
        plsc.subcore_barrier()
        row0 = wid * ROWS_PER_TILE

        @pl.loop(0, ROWS_PER_TILE // STAGE)
        def _(h):
            r0 = row0 + h * STAGE
            pltpu.sync_copy(src_hbm.at[pl.ds(r0, STAGE)], src_v)
            pltpu.sync_copy(dst_hbm.at[pl.ds(r0, STAGE)], dst_v)
            for b in range(nbuf):
                pltpu.async_copy(hs_hbm.at[src_v.at[b]], bufs[b], sg[b])

            @pl.loop(0, STAGE - nbuf, step=nbuf)
            def _(j):
                for b in range(nbuf):
                    pltpu.make_async_copy(
                        hs_hbm.at[src_v.at[j + b]], bufs[b], sg[b]).wait()
                    pltpu.async_copy(
                        bufs[b], accs[b % NACC].at[dst_v.at[j + b]],
                        ss[b], add=True)
                for b in range(nbuf):
                    pltpu.make_async_copy(
                        bufs[b], accs[b % NACC].at[dst_v.at[j + b]],
                        ss[b]).wait()
                    pltpu.async_copy(
                        hs_hbm.at[src_v.at[j + nbuf + b]], bufs[b], sg[b])

            j0 = STAGE - nbuf
            for b in range(nbuf):
                pltpu.make_async_copy(
                    hs_hbm.at[src_v.at[j0 + b]], bufs[b], sg[b]).wait()
                pltpu.async_copy(
                    bufs[b], accs[b % NACC].at[dst_v.at[j0 + b]],
                    ss[b], add=True)
            for b in range(nbuf):
                pltpu.make_async_copy(
                    bufs[b], accs[b % NACC].at[dst_v.at[j0 + b]],
                    ss[b]).wait()

        plsc.subcore_barrier()
        for a in range(NACC):
            pltpu.sync_copy(accs[a].at[pl.ds(s * rz, rz)],
                            out_hbm.at[c, a, pl.ds(s * rz, rz)])

    return edge_pass


_R = 1000


def _dinv_col(degp_ref):
    p = degp_ref[0] + degp_ref[1]
    return lax.rsqrt(1.0 + p)


def _tc_hs1_body(x_ref, w_ref, degp_ref, o_ref):
    dinv = _dinv_col(degp_ref)
    o_ref[...] = dinv * jnp.dot(x_ref[...], w_ref[...],
                                preferred_element_type=jnp.float32)


def _tc_mid_body(aggp_ref, hs_ref, degp_ref, b_ref, w_ref, o_ref):
    dinv = _dinv_col(degp_ref)
    agg = jnp.sum(aggp_ref[...].astype(jnp.float32), axis=0)
    t = dinv * (agg + hs_ref[...]) + b_ref[...]
    out1 = jnp.maximum(t, 0.0)
    o_ref[...] = dinv * jnp.dot(out1, w_ref[...],
                                preferred_element_type=jnp.float32)


def _tc_fin_body(aggp_ref, hs_ref, degp_ref, b_ref, o_ref):
    dinv = _dinv_col(degp_ref)
    agg = jnp.sum(aggp_ref[...], axis=0)[:, :D_OUT]
    z = dinv * (agg + hs_ref[...]) + b_ref[...]
    m = jnp.max(z, axis=1, keepdims=True)
    e = jnp.exp(z - m)
    lse = jnp.log(jnp.sum(e, axis=1, keepdims=True)) + m
    o_ref[...] = z - lse


def _tc_hs1(x, W1, degp):
    return pl.pallas_call(
        _tc_hs1_body,
        grid=(N // _R,),
        in_specs=[
            pl.BlockSpec((_R, D_IN), lambda i: (i, 0)),
            pl.BlockSpec((D_IN, D_H), lambda i: (0, 0)),
            pl.BlockSpec((NC, _R, 1), lambda i: (0, i, 0)),
        ],
        out_specs=pl.BlockSpec((_R, D_H), lambda i: (i, 0)),
        out_shape=jax.ShapeDtypeStruct((N, D_H), jnp.float32),
    )(x, W1, degp)


def _tc_mid(agg1p, hs1, degp, b1, W2):
    return pl.pallas_call(
        _tc_mid_body,
        grid=(N // _R,),
        in_specs=[
            pl.BlockSpec((agg1p.shape[0], _R, D_H), lambda i: (0, i, 0)),
            pl.BlockSpec((_R, D_H), lambda i: (i, 0)),
            pl.BlockSpec((NC, _R, 1), lambda i: (0, i, 0)),
            pl.BlockSpec((1, D_H), lambda i: (0, 0)),
            pl.BlockSpec((D_H, D_OUT), lambda i: (0, 0)),
        ],
        out_specs=pl.BlockSpec((_R, D_OUT), lambda i: (i, 0)),
        out_shape=jax.ShapeDtypeStruct((N, D_OUT), jnp.float32),
    )(agg1p, hs1, degp, b1, W2)


def _tc_fin(agg2p, hs2, degp, b2):
    return pl.pallas_call(
        _tc_fin_body,
        grid=(N // _R,),
        in_specs=[
            pl.BlockSpec((agg2p.shape[0], _R, agg2p.shape[2]),
                         lambda i: (0, i, 0)),
            pl.BlockSpec((_R, D_OUT), lambda i: (i, 0)),
            pl.BlockSpec((NC, _R, 1), lambda i: (0, i, 0)),
            pl.BlockSpec((1, D_OUT), lambda i: (0, 0)),
        ],
        out_specs=pl.BlockSpec((_R, D_OUT), lambda i: (i, 0)),
        out_shape=jax.ShapeDtypeStruct((N, D_OUT), jnp.float32),
    )(agg2p, hs2, degp, b2)


def kernel(x, edge_index, W1, b1, W2, b2):
    src = edge_index[0].astype(jnp.int32)
    dst = edge_index[1].astype(jnp.int32)
    pad = jnp.full((EPAD - E,), N, jnp.int32)
    srcp = jnp.concatenate([src, pad]).reshape(EROWS, SUB)
    dstp = jnp.concatenate([dst, pad]).reshape(EROWS, SUB)

    zeros_h = jnp.zeros((NPAD, D_H), jnp.float32)

    degp = _get_deg_pass()(dstp)[:, :, None]

    hs1 = _tc_hs1(x, W1, degp)
    hs1bf = jnp.concatenate(
        [hs1.astype(jnp.bfloat16),
         jnp.zeros((NPAD - N, D_H), jnp.bfloat16)], axis=0)
    zeros_bf = jnp.zeros((NPAD, D_H), jnp.bfloat16)
    agg1p = _make_edge_pass_bf16(D_H)(hs1bf, srcp, dstp, zeros_bf)
    agg1p = agg1p.reshape(NC * NACC, NPAD, D_H)

    hs2 = _tc_mid(agg1p, hs1, degp, b1.reshape(1, D_H), W2)
    hs2p = jnp.concatenate(
        [hs2, jnp.zeros((NPAD - N, D_OUT), jnp.float32)], axis=0)
    agg2p = _make_edge_pass(D_OUT, EDGE_CORES, EDGE_RPT, 4)(
        hs2p, srcp, dstp, zeros_h[:, :D_OUT])

    return _tc_fin(agg2p, hs2, degp, b2.reshape(1, D_OUT))


# speedup vs baseline: 1.4041x; 1.1392x over previous
import functools

import jax
import jax.numpy as jnp
from jax import lax
from jax.experimental import pallas as pl
from jax.experimental.pallas import tpu as pltpu
from jax.experimental.pallas import tpu_sc as plsc

N = 10000
E = 320000
D_IN = 128
D_H = 128
D_OUT = 64

NC = 2
NS = 16
NW = NC * NS

SUB = 128
KROWS = 8
ROWS_PER_TILE = 80
OUTER = ROWS_PER_TILE // KROWS
EPAD = NW * ROWS_PER_TILE * SUB
EROWS = EPAD // SUB
NPAD = EPAD // NW
L = 16

@functools.cache
def _get_deg_pass():
    mesh = plsc.VectorSubcoreMesh(core_axis_name="c", subcore_axis_name="s")
    rz = NPAD // NS

    @functools.partial(
        pl.kernel,
        out_type=jax.ShapeDtypeStruct((NC, NPAD), jnp.float32),
        mesh=mesh,
        scratch_types=[
            pltpu.VMEM((ROWS_PER_TILE, SUB), jnp.int32),
            pltpu.VMEM((NPAD,), jnp.float32),
            pltpu.VMEM((NS, rz), jnp.float32),
            pltpu.VMEM((rz,), jnp.float32),
            pltpu.VMEM_SHARED((NS, NPAD), jnp.float32),
        ],
        compiler_params=pltpu.CompilerParams(needs_layout_passes=False),
    )
    def _deg_pass(dst_hbm, out_hbm, dst_v, acc_v, red_v, out_v, sh):
        c = lax.axis_index("c")
        s = lax.axis_index("s")
        wid = c * NS + s
        row0 = wid * ROWS_PER_TILE
        pltpu.sync_copy(dst_hbm.at[pl.ds(row0, ROWS_PER_TILE)], dst_v)

        zeros = jnp.zeros((L,), jnp.float32)

        @pl.loop(0, NPAD, step=L)
        def _(j):
            acc_v[pl.ds(j, L)] = zeros

        ones = jnp.ones((L,), jnp.float32)

        @pl.loop(0, ROWS_PER_TILE)
        def _(r):
            for k in range(SUB // L):
                idx = dst_v[r, pl.ds(k * L, L)]
                plsc.addupdate_scatter(acc_v, [idx], ones)

        pltpu.sync_copy(acc_v, sh.at[s])
        plsc.subcore_barrier()
        for r in range(NS):
            pltpu.sync_copy(sh.at[r, pl.ds(s * rz, rz)], red_v.at[r])

        @pl.loop(0, rz, step=L)
        def _(j):
            v = red_v[0, pl.ds(j, L)]
            for r in range(1, NS):
                v = v + red_v[r, pl.ds(j, L)]
            out_v[pl.ds(j, L)] = v

        pltpu.sync_copy(out_v, out_hbm.at[c, pl.ds(s * rz, rz)])

    return _deg_pass


NBUF = 2
STAGE = 40
EDGE_CORES = 2
EDGE_RPT = EROWS // (EDGE_CORES * NS)


@functools.cache
def _make_edge_pass(D, ncores=NC, rows_per_tile=ROWS_PER_TILE, nbuf=NBUF):
    mesh = plsc.VectorSubcoreMesh(
        core_axis_name="c", subcore_axis_name="s", num_cores=ncores)

    @functools.partial(
        pl.kernel,
        out_type=jax.ShapeDtypeStruct((ncores, NPAD, D), jnp.float32),
        mesh=mesh,
        scratch_types=[
            pltpu.VMEM((STAGE, SUB), jnp.int32),
            pltpu.VMEM((STAGE, SUB), jnp.int32),
        ]
        + [pltpu.VMEM((SUB, D), jnp.float32) for _ in range(nbuf)]
        + [pltpu.SemaphoreType.DMA for _ in range(2 * nbuf)]
        + [pltpu.VMEM_SHARED((NPAD, D), jnp.float32)],
        compiler_params=(
            pltpu.CompilerParams(use_tc_tiling_on_sc=False)
            if D % 128 != 0 else None),
    )
    def edge_pass(hs_hbm, src_hbm, dst_hbm, zeros_hbm, out_hbm,
                  src_v, dst_v, *rest):
        bufs = rest[:nbuf]
        sg = rest[nbuf:2 * nbuf]
        ss = rest[2 * nbuf:3 * nbuf]
        acc = rest[3 * nbuf]
        c = lax.axis_index("c")
        s = lax.axis_index("s")
        wid = c * NS + s
        rz = NPAD // NS
        pltpu.sync_copy(zeros_hbm.at[pl.ds(s * rz, rz)],
                        acc.at[pl.ds(s * rz, rz)])
        plsc.subcore_barrier()
        row0 = wid * rows_per_tile

        @pl.loop(0, rows_per_tile // STAGE)
        def _(h):
            r0 = row0 + h * STAGE
            pltpu.sync_copy(src_hbm.at[pl.ds(r0, STAGE)], src_v)
            pltpu.sync_copy(dst_hbm.at[pl.ds(r0, STAGE)], dst_v)
            for b in range(nbuf):
                pltpu.async_copy(hs_hbm.at[src_v.at[b]], bufs[b], sg[b])

            @pl.loop(0, STAGE - nbuf, step=nbuf)
            def _(j):
                for b in range(nbuf):
                    pltpu.make_async_copy(
                        hs_hbm.at[src_v.at[j + b]], bufs[b], sg[b]).wait()
                    pltpu.async_copy(
                        bufs[b], acc.at[dst_v.at[j + b]], ss[b], add=True)
                for b in range(nbuf):
                    pltpu.make_async_copy(
                        bufs[b], acc.at[dst_v.at[j + b]], ss[b]).wait()
                    pltpu.async_copy(
                        hs_hbm.at[src_v.at[j + nbuf + b]], bufs[b], sg[b])

            j0 = STAGE - nbuf
            for b in range(nbuf):
                pltpu.make_async_copy(
                    hs_hbm.at[src_v.at[j0 + b]], bufs[b], sg[b]).wait()
                pltpu.async_copy(
                    bufs[b], acc.at[dst_v.at[j0 + b]], ss[b], add=True)
            for b in range(nbuf):
                pltpu.make_async_copy(
                    bufs[b], acc.at[dst_v.at[j0 + b]], ss[b]).wait()

        plsc.subcore_barrier()
        pltpu.sync_copy(acc.at[pl.ds(s * rz, rz)],
                        out_hbm.at[c, pl.ds(s * rz, rz)])

    return edge_pass


NACC = 2


@functools.cache
def _make_edge_pass_bf16(D, nbuf=4):
    mesh = plsc.VectorSubcoreMesh(
        core_axis_name="c", subcore_axis_name="s", num_cores=NC)

    @functools.partial(
        pl.kernel,
        out_type=jax.ShapeDtypeStruct((NC, NACC, NPAD, D), jnp.bfloat16),
        mesh=mesh,
        scratch_types=[
            pltpu.VMEM((STAGE, SUB), jnp.int32),
            pltpu.VMEM((STAGE, SUB), jnp.int32),
        ]
        + [pltpu.VMEM((SUB, D), jnp.bfloat16) for _ in range(nbuf)]
        + [pltpu.SemaphoreType.DMA for _ in range(2 * nbuf)]
        + [pltpu.VMEM_SHARED((NPAD, D), jnp.bfloat16) for _ in range(NACC)],
        compiler_params=pltpu.CompilerParams(use_tc_tiling_on_sc=False),
    )
    def edge_pass(hs_hbm, src_hbm, dst_hbm, zeros_hbm, out_hbm,
                  src_v, dst_v, *rest):
        bufs = rest[:nbuf]
        sg = rest[nbuf:2 * nbuf]
        ss = rest[2 * nbuf:3 * nbuf]
        accs = rest[3 * nbuf:3 * nbuf + NACC]
        c = lax.axis_index("c")
        s = lax.axis_index("s")
        wid = c * NS + s
        rz = NPAD // NS
        for a in range(NACC):
            pltpu.sync_copy(zeros_hbm.at[pl.ds(s * rz, rz)],
                            accs[a].at[pl.ds(s * rz, rz)])
        plsc.subcore_barrier()
        row0 = wid * ROWS_PER_TILE

        @pl.loop(0, ROWS_PER_TILE // STAGE)
        def _(h):
            r0 = row0 + h * STAGE
            pltpu.sync_copy(src_hbm.at[pl.ds(r0, STAGE)], src_v)
            pltpu.sync_copy(dst_hbm.at[pl.ds(r0, STAGE)], dst_v)
            for b in range(nbuf):
                pltpu.async_copy(hs_hbm.at[src_v.at[b]], bufs[b], sg[b])

            @pl.loop(0, STAGE - nbuf, step=nbuf)
            def _(j):
                for b in range(nbuf):
                    pltpu.make_async_copy(
                        hs_hbm.at[src_v.at[j + b]], bufs[b], sg[b]).wait()
                    pltpu.async_copy(
                        bufs[b], accs[b % NACC].at[dst_v.at[j + b]],
                        ss[b], add=True)
                for b in range(nbuf):
                    pltpu.make_async_copy(
                        bufs[b], accs[b % NACC].at[dst_v.at[j + b]],
                        ss[b]).wait()
                    pltpu.async_copy(
                        hs_hbm.at[src_v.at[j + nbuf + b]], bufs[b], sg[b])

            j0 = STAGE - nbuf
            for b in range(nbuf):
                pltpu.make_async_copy(
                    hs_hbm.at[src_v.at[j0 + b]], bufs[b], sg[b]).wait()
                pltpu.async_copy(
                    bufs[b], accs[b % NACC].at[dst_v.at[j0 + b]],
                    ss[b], add=True)
            for b in range(nbuf):
                pltpu.make_async_copy(
                    bufs[b], accs[b % NACC].at[dst_v.at[j0 + b]],
                    ss[b]).wait()

        plsc.subcore_barrier()
        for a in range(NACC):
            pltpu.sync_copy(accs[a].at[pl.ds(s * rz, rz)],
                            out_hbm.at[c, a, pl.ds(s * rz, rz)])

    return edge_pass


_R = 1000


def _dinv_col(degp_ref):
    p = degp_ref[0] + degp_ref[1]
    return lax.rsqrt(1.0 + p)


def _tc_hs1_body(x_ref, w_ref, degp_ref, o_ref):
    dinv = _dinv_col(degp_ref)
    o_ref[...] = dinv * jnp.dot(x_ref[...], w_ref[...],
                                preferred_element_type=jnp.float32)


def _tc_mid_body(aggp_ref, hs_ref, degp_ref, b_ref, w_ref, o_ref):
    dinv = _dinv_col(degp_ref)
    agg = jnp.sum(aggp_ref[...].astype(jnp.float32), axis=0)
    t = dinv * (agg + hs_ref[...]) + b_ref[...]
    out1 = jnp.maximum(t, 0.0)
    o_ref[...] = dinv * jnp.dot(out1, w_ref[...],
                                preferred_element_type=jnp.float32)


def _tc_fin_body(aggp_ref, hs_ref, degp_ref, b_ref, o_ref):
    dinv = _dinv_col(degp_ref)
    agg = jnp.sum(aggp_ref[...].astype(jnp.float32), axis=0)[:, :D_OUT]
    z = dinv * (agg + hs_ref[...]) + b_ref[...]
    m = jnp.max(z, axis=1, keepdims=True)
    e = jnp.exp(z - m)
    lse = jnp.log(jnp.sum(e, axis=1, keepdims=True)) + m
    o_ref[...] = z - lse


def _tc_hs1(x, W1, degp):
    return pl.pallas_call(
        _tc_hs1_body,
        grid=(N // _R,),
        in_specs=[
            pl.BlockSpec((_R, D_IN), lambda i: (i, 0)),
            pl.BlockSpec((D_IN, D_H), lambda i: (0, 0)),
            pl.BlockSpec((NC, _R, 1), lambda i: (0, i, 0)),
        ],
        out_specs=pl.BlockSpec((_R, D_H), lambda i: (i, 0)),
        out_shape=jax.ShapeDtypeStruct((N, D_H), jnp.float32),
    )(x, W1, degp)


def _tc_mid(agg1p, hs1, degp, b1, W2):
    return pl.pallas_call(
        _tc_mid_body,
        grid=(N // _R,),
        in_specs=[
            pl.BlockSpec((agg1p.shape[0], _R, D_H), lambda i: (0, i, 0)),
            pl.BlockSpec((_R, D_H), lambda i: (i, 0)),
            pl.BlockSpec((NC, _R, 1), lambda i: (0, i, 0)),
            pl.BlockSpec((1, D_H), lambda i: (0, 0)),
            pl.BlockSpec((D_H, D_OUT), lambda i: (0, 0)),
        ],
        out_specs=pl.BlockSpec((_R, D_OUT), lambda i: (i, 0)),
        out_shape=jax.ShapeDtypeStruct((N, D_OUT), jnp.float32),
    )(agg1p, hs1, degp, b1, W2)


def _tc_fin(agg2p, hs2, degp, b2):
    return pl.pallas_call(
        _tc_fin_body,
        grid=(N // _R,),
        in_specs=[
            pl.BlockSpec((agg2p.shape[0], _R, agg2p.shape[2]),
                         lambda i: (0, i, 0)),
            pl.BlockSpec((_R, D_OUT), lambda i: (i, 0)),
            pl.BlockSpec((NC, _R, 1), lambda i: (0, i, 0)),
            pl.BlockSpec((1, D_OUT), lambda i: (0, 0)),
        ],
        out_specs=pl.BlockSpec((_R, D_OUT), lambda i: (i, 0)),
        out_shape=jax.ShapeDtypeStruct((N, D_OUT), jnp.float32),
    )(agg2p, hs2, degp, b2)


def kernel(x, edge_index, W1, b1, W2, b2):
    src = edge_index[0].astype(jnp.int32)
    dst = edge_index[1].astype(jnp.int32)
    pad = jnp.full((EPAD - E,), N, jnp.int32)
    srcp = jnp.concatenate([src, pad]).reshape(EROWS, SUB)
    dstp = jnp.concatenate([dst, pad]).reshape(EROWS, SUB)

    degp = _get_deg_pass()(dstp)[:, :, None]

    hs1 = _tc_hs1(x, W1, degp)
    hs1bf = jnp.concatenate(
        [hs1.astype(jnp.bfloat16),
         jnp.zeros((NPAD - N, D_H), jnp.bfloat16)], axis=0)
    zeros_bf = jnp.zeros((NPAD, D_H), jnp.bfloat16)
    agg1p = _make_edge_pass_bf16(D_H)(hs1bf, srcp, dstp, zeros_bf)
    agg1p = agg1p.reshape(NC * NACC, NPAD, D_H)

    hs2 = _tc_mid(agg1p, hs1, degp, b1.reshape(1, D_H), W2)
    hs2bf = jnp.concatenate(
        [hs2.astype(jnp.bfloat16),
         jnp.zeros((NPAD - N, D_OUT), jnp.bfloat16)], axis=0)
    agg2p = _make_edge_pass_bf16(D_OUT)(hs2bf, srcp, dstp,
                                        zeros_bf[:, :D_OUT])
    agg2p = agg2p.reshape(NC * NACC, NPAD, D_OUT)

    return _tc_fin(agg2p, hs2, degp, b2.reshape(1, D_OUT))


# speedup vs baseline: 1.5197x; 1.0823x over previous
import functools

import jax
import jax.numpy as jnp
from jax import lax
from jax.experimental import pallas as pl
from jax.experimental.pallas import tpu as pltpu
from jax.experimental.pallas import tpu_sc as plsc

N = 10000
E = 320000
D_IN = 128
D_H = 128
D_OUT = 64

NC = 2
NS = 16
NW = NC * NS

SUB = 128
KROWS = 8
ROWS_PER_TILE = 80
OUTER = ROWS_PER_TILE // KROWS
EPAD = NW * ROWS_PER_TILE * SUB
EROWS = EPAD // SUB
NPAD = EPAD // NW
L = 16

@functools.cache
def _get_deg_pass():
    mesh = plsc.VectorSubcoreMesh(core_axis_name="c", subcore_axis_name="s")
    rz = NPAD // NS

    @functools.partial(
        pl.kernel,
        out_type=jax.ShapeDtypeStruct((NC, NPAD), jnp.float32),
        mesh=mesh,
        scratch_types=[
            pltpu.VMEM((ROWS_PER_TILE, SUB), jnp.int32),
            pltpu.VMEM((NPAD,), jnp.float32),
            pltpu.VMEM((NS, rz), jnp.float32),
            pltpu.VMEM((rz,), jnp.float32),
            pltpu.VMEM_SHARED((NS, NPAD), jnp.float32),
        ],
        compiler_params=pltpu.CompilerParams(needs_layout_passes=False),
    )
    def _deg_pass(dst_hbm, out_hbm, dst_v, acc_v, red_v, out_v, sh):
        c = lax.axis_index("c")
        s = lax.axis_index("s")
        wid = c * NS + s
        row0 = wid * ROWS_PER_TILE
        pltpu.sync_copy(dst_hbm.at[pl.ds(row0, ROWS_PER_TILE)], dst_v)

        zeros = jnp.zeros((L,), jnp.float32)

        @pl.loop(0, NPAD, step=L)
        def _(j):
            acc_v[pl.ds(j, L)] = zeros

        ones = jnp.ones((L,), jnp.float32)

        @pl.loop(0, ROWS_PER_TILE)
        def _(r):
            for k in range(SUB // L):
                idx = dst_v[r, pl.ds(k * L, L)]
                plsc.addupdate_scatter(acc_v, [idx], ones)

        pltpu.sync_copy(acc_v, sh.at[s])
        plsc.subcore_barrier()
        for r in range(NS):
            pltpu.sync_copy(sh.at[r, pl.ds(s * rz, rz)], red_v.at[r])

        @pl.loop(0, rz, step=L)
        def _(j):
            v = red_v[0, pl.ds(j, L)]
            for r in range(1, NS):
                v = v + red_v[r, pl.ds(j, L)]
            out_v[pl.ds(j, L)] = v

        pltpu.sync_copy(out_v, out_hbm.at[c, pl.ds(s * rz, rz)])

    return _deg_pass


NBUF = 2
STAGE = 40
EDGE_CORES = 2
EDGE_RPT = EROWS // (EDGE_CORES * NS)


@functools.cache
def _make_edge_pass(D, ncores=NC, rows_per_tile=ROWS_PER_TILE, nbuf=NBUF):
    mesh = plsc.VectorSubcoreMesh(
        core_axis_name="c", subcore_axis_name="s", num_cores=ncores)

    @functools.partial(
        pl.kernel,
        out_type=jax.ShapeDtypeStruct((ncores, NPAD, D), jnp.float32),
        mesh=mesh,
        scratch_types=[
            pltpu.VMEM((STAGE, SUB), jnp.int32),
            pltpu.VMEM((STAGE, SUB), jnp.int32),
        ]
        + [pltpu.VMEM((SUB, D), jnp.float32) for _ in range(nbuf)]
        + [pltpu.SemaphoreType.DMA for _ in range(2 * nbuf)]
        + [pltpu.VMEM_SHARED((NPAD, D), jnp.float32)],
        compiler_params=(
            pltpu.CompilerParams(use_tc_tiling_on_sc=False)
            if D % 128 != 0 else None),
    )
    def edge_pass(hs_hbm, src_hbm, dst_hbm, zeros_hbm, out_hbm,
                  src_v, dst_v, *rest):
        bufs = rest[:nbuf]
        sg = rest[nbuf:2 * nbuf]
        ss = rest[2 * nbuf:3 * nbuf]
        acc = rest[3 * nbuf]
        c = lax.axis_index("c")
        s = lax.axis_index("s")
        wid = c * NS + s
        rz = NPAD // NS
        pltpu.sync_copy(zeros_hbm.at[pl.ds(s * rz, rz)],
                        acc.at[pl.ds(s * rz, rz)])
        plsc.subcore_barrier()
        row0 = wid * rows_per_tile

        @pl.loop(0, rows_per_tile // STAGE)
        def _(h):
            r0 = row0 + h * STAGE
            pltpu.sync_copy(src_hbm.at[pl.ds(r0, STAGE)], src_v)
            pltpu.sync_copy(dst_hbm.at[pl.ds(r0, STAGE)], dst_v)
            for b in range(nbuf):
                pltpu.async_copy(hs_hbm.at[src_v.at[b]], bufs[b], sg[b])

            @pl.loop(0, STAGE - nbuf, step=nbuf)
            def _(j):
                for b in range(nbuf):
                    pltpu.make_async_copy(
                        hs_hbm.at[src_v.at[j + b]], bufs[b], sg[b]).wait()
                    pltpu.async_copy(
                        bufs[b], acc.at[dst_v.at[j + b]], ss[b], add=True)
                for b in range(nbuf):
                    pltpu.make_async_copy(
                        bufs[b], acc.at[dst_v.at[j + b]], ss[b]).wait()
                    pltpu.async_copy(
                        hs_hbm.at[src_v.at[j + nbuf + b]], bufs[b], sg[b])

            j0 = STAGE - nbuf
            for b in range(nbuf):
                pltpu.make_async_copy(
                    hs_hbm.at[src_v.at[j0 + b]], bufs[b], sg[b]).wait()
                pltpu.async_copy(
                    bufs[b], acc.at[dst_v.at[j0 + b]], ss[b], add=True)
            for b in range(nbuf):
                pltpu.make_async_copy(
                    bufs[b], acc.at[dst_v.at[j0 + b]], ss[b]).wait()

        plsc.subcore_barrier()
        pltpu.sync_copy(acc.at[pl.ds(s * rz, rz)],
                        out_hbm.at[c, pl.ds(s * rz, rz)])

    return edge_pass


NACC = 2


@functools.cache
def _make_edge_pass_bf16(D, nbuf=4):
    mesh = plsc.VectorSubcoreMesh(
        core_axis_name="c", subcore_axis_name="s", num_cores=NC)

    @functools.partial(
        pl.kernel,
        out_type=jax.ShapeDtypeStruct((NC, NACC, NPAD, D), jnp.bfloat16),
        mesh=mesh,
        scratch_types=[
            pltpu.VMEM((STAGE, SUB), jnp.int32),
            pltpu.VMEM((STAGE, SUB), jnp.int32),
        ]
        + [pltpu.VMEM((SUB, D), jnp.bfloat16) for _ in range(nbuf)]
        + [pltpu.SemaphoreType.DMA for _ in range(2 * nbuf)]
        + [pltpu.VMEM_SHARED((NPAD, D), jnp.bfloat16) for _ in range(NACC)],
        compiler_params=pltpu.CompilerParams(use_tc_tiling_on_sc=False),
    )
    def edge_pass(hs_hbm, src_hbm, dst_hbm, zeros_hbm, out_hbm,
                  src_v, dst_v, *rest):
        bufs = rest[:nbuf]
        sg = rest[nbuf:2 * nbuf]
        ss = rest[2 * nbuf:3 * nbuf]
        accs = rest[3 * nbuf:3 * nbuf + NACC]
        c = lax.axis_index("c")
        s = lax.axis_index("s")
        wid = c * NS + s
        rz = NPAD // NS
        for a in range(NACC):
            pltpu.sync_copy(zeros_hbm.at[pl.ds(s * rz, rz)],
                            accs[a].at[pl.ds(s * rz, rz)])
        plsc.subcore_barrier()
        row0 = wid * ROWS_PER_TILE

        @pl.loop(0, ROWS_PER_TILE // STAGE)
        def _(h):
            r0 = row0 + h * STAGE
            pltpu.sync_copy(src_hbm.at[pl.ds(r0, STAGE)], src_v)
            pltpu.sync_copy(dst_hbm.at[pl.ds(r0, STAGE)], dst_v)
            for b in range(nbuf):
                pltpu.async_copy(hs_hbm.at[src_v.at[b]], bufs[b], sg[b])

            @pl.loop(0, STAGE - nbuf, step=nbuf)
            def _(j):
                for b in range(nbuf):
                    pltpu.make_async_copy(
                        hs_hbm.at[src_v.at[j + b]], bufs[b], sg[b]).wait()
                    pltpu.async_copy(
                        bufs[b], accs[b % NACC].at[dst_v.at[j + b]],
                        ss[b], add=True)
                for b in range(nbuf):
                    pltpu.make_async_copy(
                        bufs[b], accs[b % NACC].at[dst_v.at[j + b]],
                        ss[b]).wait()
                    pltpu.async_copy(
                        hs_hbm.at[src_v.at[j + nbuf + b]], bufs[b], sg[b])

            j0 = STAGE - nbuf
            for b in range(nbuf):
                pltpu.make_async_copy(
                    hs_hbm.at[src_v.at[j0 + b]], bufs[b], sg[b]).wait()
                pltpu.async_copy(
                    bufs[b], accs[b % NACC].at[dst_v.at[j0 + b]],
                    ss[b], add=True)
            for b in range(nbuf):
                pltpu.make_async_copy(
                    bufs[b], accs[b % NACC].at[dst_v.at[j0 + b]],
                    ss[b]).wait()

        plsc.subcore_barrier()
        for a in range(NACC):
            pltpu.sync_copy(accs[a].at[pl.ds(s * rz, rz)],
                            out_hbm.at[c, a, pl.ds(s * rz, rz)])

    return edge_pass


_R = 1000


def _dinv_col(degp_ref):
    p = degp_ref[0] + degp_ref[1]
    return lax.rsqrt(1.0 + p)


def _tc_hs1_body(x_ref, w_ref, degp_ref, o_ref):
    dinv = _dinv_col(degp_ref)
    o_ref[...] = dinv * jnp.dot(x_ref[...], w_ref[...],
                                preferred_element_type=jnp.float32)


def _tc_mid_body_bf16(aggp_ref, hs_ref, degp_ref, b_ref, w_ref, o_ref):
    dinv = _dinv_col(degp_ref)
    agg = jnp.sum(aggp_ref[...].astype(jnp.float32), axis=0)
    t = dinv * (agg + hs_ref[...]) + b_ref[...]
    out1 = jnp.maximum(t, 0.0).astype(jnp.bfloat16)
    o_ref[...] = dinv * jnp.dot(out1, w_ref[...],
                                preferred_element_type=jnp.float32)


def _tc_mid_body(aggp_ref, hs_ref, degp_ref, b_ref, w_ref, o_ref):
    dinv = _dinv_col(degp_ref)
    agg = jnp.sum(aggp_ref[...].astype(jnp.float32), axis=0)
    t = dinv * (agg + hs_ref[...]) + b_ref[...]
    out1 = jnp.maximum(t, 0.0)
    o_ref[...] = dinv * jnp.dot(out1, w_ref[...],
                                preferred_element_type=jnp.float32)


def _tc_fin_body(aggp_ref, hs_ref, degp_ref, b_ref, o_ref):
    dinv = _dinv_col(degp_ref)
    agg = jnp.sum(aggp_ref[...].astype(jnp.float32), axis=0)[:, :D_OUT]
    z = dinv * (agg + hs_ref[...]) + b_ref[...]
    m = jnp.max(z, axis=1, keepdims=True)
    e = jnp.exp(z - m)
    lse = jnp.log(jnp.sum(e, axis=1, keepdims=True)) + m
    o_ref[...] = z - lse


def _tc_hs1(x, W1, degp):
    return pl.pallas_call(
        _tc_hs1_body,
        grid=(N // _R,),
        in_specs=[
            pl.BlockSpec((_R, D_IN), lambda i: (i, 0)),
            pl.BlockSpec((D_IN, D_H), lambda i: (0, 0)),
            pl.BlockSpec((NC, _R, 1), lambda i: (0, i, 0)),
        ],
        out_specs=pl.BlockSpec((_R, D_H), lambda i: (i, 0)),
        out_shape=jax.ShapeDtypeStruct((N, D_H), jnp.float32),
    )(x, W1, degp)


def _tc_mid(agg1p, hs1, degp, b1, W2):
    return pl.pallas_call(
        _tc_mid_body_bf16 if W2.dtype == jnp.bfloat16 else _tc_mid_body,
        grid=(N // _R,),
        in_specs=[
            pl.BlockSpec((agg1p.shape[0], _R, D_H), lambda i: (0, i, 0)),
            pl.BlockSpec((_R, D_H), lambda i: (i, 0)),
            pl.BlockSpec((NC, _R, 1), lambda i: (0, i, 0)),
            pl.BlockSpec((1, D_H), lambda i: (0, 0)),
            pl.BlockSpec((D_H, D_OUT), lambda i: (0, 0)),
        ],
        out_specs=pl.BlockSpec((_R, D_OUT), lambda i: (i, 0)),
        out_shape=jax.ShapeDtypeStruct((N, D_OUT), jnp.float32),
    )(agg1p, hs1, degp, b1, W2)


def _tc_fin(agg2p, hs2, degp, b2):
    return pl.pallas_call(
        _tc_fin_body,
        grid=(N // _R,),
        in_specs=[
            pl.BlockSpec((agg2p.shape[0], _R, agg2p.shape[2]),
                         lambda i: (0, i, 0)),
            pl.BlockSpec((_R, D_OUT), lambda i: (i, 0)),
            pl.BlockSpec((NC, _R, 1), lambda i: (0, i, 0)),
            pl.BlockSpec((1, D_OUT), lambda i: (0, 0)),
        ],
        out_specs=pl.BlockSpec((_R, D_OUT), lambda i: (i, 0)),
        out_shape=jax.ShapeDtypeStruct((N, D_OUT), jnp.float32),
    )(agg2p, hs2, degp, b2)


def kernel(x, edge_index, W1, b1, W2, b2):
    src = edge_index[0].astype(jnp.int32)
    dst = edge_index[1].astype(jnp.int32)
    pad = jnp.full((EPAD - E,), N, jnp.int32)
    srcp = jnp.concatenate([src, pad]).reshape(EROWS, SUB)
    dstp = jnp.concatenate([dst, pad]).reshape(EROWS, SUB)

    degp = _get_deg_pass()(dstp)[:, :, None]

    hs1 = _tc_hs1(x.astype(jnp.bfloat16), W1.astype(jnp.bfloat16), degp)
    hs1bf = jnp.concatenate(
        [hs1.astype(jnp.bfloat16),
         jnp.zeros((NPAD - N, D_H), jnp.bfloat16)], axis=0)
    zeros_bf = jnp.zeros((NPAD, D_H), jnp.bfloat16)
    agg1p = _make_edge_pass_bf16(D_H)(hs1bf, srcp, dstp, zeros_bf)
    agg1p = agg1p.reshape(NC * NACC, NPAD, D_H)

    hs2 = _tc_mid(agg1p, hs1, degp, b1.reshape(1, D_H),
                  W2.astype(jnp.bfloat16))
    hs2bf = jnp.concatenate(
        [hs2.astype(jnp.bfloat16),
         jnp.zeros((NPAD - N, D_OUT), jnp.bfloat16)], axis=0)
    agg2p = _make_edge_pass_bf16(D_OUT)(hs2bf, srcp, dstp,
                                        zeros_bf[:, :D_OUT])
    agg2p = agg2p.reshape(NC * NACC, NPAD, D_OUT)

    return _tc_fin(agg2p, hs2, degp, b2.reshape(1, D_OUT))
